# Initial kernel scaffold; baseline (speedup 1.0000x reference)
import jax
import jax.numpy as jnp
from jax.experimental import pallas as pl


def kernel(x, edge_index, batch, params):
    raise NotImplementedError("write your pallas kernel here")


# speedup vs baseline: 1.0887x; 1.0887x over previous
import functools
import math

import jax
import jax.numpy as jnp
from jax.experimental import pallas as pl
from jax.experimental.pallas import tpu as pltpu

N_NODES = 10000
RATIO = 0.8


def _mm_kernel(a_ref, b_ref, o_ref):
    o_ref[...] = jnp.dot(a_ref[...], b_ref[...],
                         preferred_element_type=jnp.float32)


def _mm(a, b, bm=512):
    m, k = a.shape
    _, n = b.shape
    grid = (pl.cdiv(m, bm),)
    return pl.pallas_call(
        _mm_kernel,
        grid=grid,
        in_specs=[
            pl.BlockSpec((bm, k), lambda i: (i, 0)),
            pl.BlockSpec((k, n), lambda i: (0, 0)),
        ],
        out_specs=pl.BlockSpec((bm, n), lambda i: (i, 0)),
        out_shape=jax.ShapeDtypeStruct((m, n), jnp.float32),
    )(a, b)


def _gat_conv(x, src, dst, emask, prm):
    N = x.shape[0]
    h = _mm(x, prm['W'])
    loop = jnp.arange(N, dtype=src.dtype)
    s = jnp.concatenate([src, loop])
    d = jnp.concatenate([dst, loop])
    m = jnp.concatenate([emask, jnp.ones((N,), x.dtype)])
    al = h @ prm['a_src']
    be = h @ prm['a_dst']
    e = jax.nn.leaky_relu(al[s] + be[d], 0.2)
    e = jnp.where(m > 0, e, -1e9)
    emax = jax.ops.segment_max(e, d, num_segments=N)
    ee = jnp.exp(e - emax[d]) * m
    denom = jax.ops.segment_sum(ee, d, num_segments=N)
    coef = ee / (denom[d] + 1e-16)
    out = jax.ops.segment_sum(h[s] * coef[:, None], d, num_segments=N)
    return out + prm['b']


def _graph_conv(x, src, dst, emask, prm):
    agg = jax.ops.segment_sum(x[src] * emask[:, None], dst, num_segments=x.shape[0])
    return _mm(agg, prm['W_rel']) + _mm(x, prm['W_root']) + prm['b']


def _topk_pool(x, src, dst, emask, batch, p):
    N = x.shape[0]
    k = int(math.ceil(RATIO * N))
    score = (x @ p) / (jnp.linalg.norm(p) + 1e-16)
    vals, perm = jax.lax.top_k(score, k)
    xk = x[perm] * jnp.tanh(vals)[:, None]
    bk = batch[perm]
    new_idx = jnp.full((N,), -1, dtype=src.dtype).at[perm].set(jnp.arange(k, dtype=src.dtype))
    ns = new_idx[src]
    nd = new_idx[dst]
    valid = (ns >= 0) & (nd >= 0) & (emask > 0)
    ns = jnp.where(valid, ns, 0)
    nd = jnp.where(valid, nd, 0)
    return xk, ns, nd, valid.astype(x.dtype), bk


def _readout(x, batch):
    mx = jax.ops.segment_max(x, batch, num_segments=1)
    sm = jax.ops.segment_sum(x, batch, num_segments=1)
    cnt = jax.ops.segment_sum(jnp.ones((x.shape[0],), x.dtype), batch, num_segments=1)
    return jnp.concatenate([mx, sm / cnt[:, None]], axis=1)


def kernel(x, edge_index, batch, params):
    src = edge_index[0]
    dst = edge_index[1]
    m0 = jnp.ones((src.shape[0],), x.dtype)
    x0 = jax.nn.relu(_gat_conv(x, src, dst, m0, params['gat10']))
    x0, s, d, m, b = _topk_pool(x0, src, dst, m0, batch, params['pool20'])
    r1 = _readout(x0, b)
    x0 = jax.nn.relu(_gat_conv(x0, s, d, m, params['gat20']))
    x0, s, d, m, b = _topk_pool(x0, s, d, m, b, params['pool20'])
    r2 = _readout(x0, b)
    x0 = jax.nn.relu(_gat_conv(x0, s, d, m, params['gat30']))
    x0, s, d, m, b = _topk_pool(x0, s, d, m, b, params['pool30'])
    r3 = _readout(x0, b)
    z = jax.nn.relu(_graph_conv(x, src, dst, m0, params['gc11']))
    z, s2, d2, m2, b2 = _topk_pool(z, src, dst, m0, batch, params['pool11'])
    z1 = _readout(z, b2)
    z = jax.nn.relu(_graph_conv(z, s2, d2, m2, params['gc21']))
    z, s2, d2, m2, b2 = _topk_pool(z, s2, d2, m2, b2, params['pool21'])
    z2 = _readout(z, b2)
    z = jax.nn.relu(_graph_conv(z, s2, d2, m2, params['gc31']))
    z, s2, d2, m2, b2 = _topk_pool(z, s2, d2, m2, b2, params['pool31'])
    z3 = _readout(z, b2)
    h = r1 + r2 + r3 + z1 + z2 + z3
    h = jax.nn.relu(h @ params['lin1']['W'] + params['lin1']['b'])
    h = jax.nn.leaky_relu(h @ params['lin2']['W'] + params['lin2']['b'], 0.01)
    h = h @ params['lin3']['W'] + params['lin3']['b']
    return jax.nn.log_softmax(h, axis=-1)
